# Initial kernel scaffold; baseline (speedup 1.0000x reference)
import functools

import jax
import jax.numpy as jnp
from jax import lax
from jax.experimental import pallas as pl
from jax.experimental.pallas import tpu as pltpu
from jax.experimental.pallas import tpu_sc as plsc

NC = 2
NS = 16
L = 16
NW = NC * NS
BE = 1024
ITERS = 6

f32 = jnp.float32


def _gelu(v):
    return jax.nn.gelu(v, approximate=True)


def _mesh():
    return plsc.VectorSubcoreMesh(
        core_axis_name="c", subcore_axis_name="s", num_cores=NC, num_subcores=NS
    )


def _wid():
    return lax.axis_index("s") * NC + lax.axis_index("c")


def _make_rad_kernel(n, e_pad):
    epw = e_pad // NW

    @functools.partial(
        pl.kernel,
        out_type=jax.ShapeDtypeStruct((e_pad,), f32),
        mesh=_mesh(),
        scratch_types=[
            pltpu.VMEM((n,), f32),
            pltpu.VMEM((n,), f32),
            pltpu.VMEM((n,), f32),
            pltpu.VMEM((epw,), jnp.int32),
            pltpu.VMEM((epw,), jnp.int32),
            pltpu.VMEM((epw,), f32),
        ],
    )
    def rad_kernel(px_h, py_h, pz_h, src_h, dst_h, rad_h, px, py, pz, si, di, ro):
        base = _wid() * epw
        pltpu.sync_copy(px_h, px)
        pltpu.sync_copy(py_h, py)
        pltpu.sync_copy(pz_h, pz)
        pltpu.sync_copy(src_h.at[pl.ds(base, epw)], si)
        pltpu.sync_copy(dst_h.at[pl.ds(base, epw)], di)

        @plsc.parallel_loop(0, epw // L, unroll=4)
        def _(v):
            s = si[pl.ds(v * L, L)]
            d = di[pl.ds(v * L, L)]
            dx = plsc.load_gather(px, [s]) - plsc.load_gather(px, [d])
            dy = plsc.load_gather(py, [s]) - plsc.load_gather(py, [d])
            dz = plsc.load_gather(pz, [s]) - plsc.load_gather(pz, [d])
            ro[pl.ds(v * L, L)] = dx * dx + dy * dy + dz * dz

        pltpu.sync_copy(ro, rad_h.at[pl.ds(base, epw)])

    return rad_kernel


def _make_gather_kernel(n_pad, e_pad, h):
    epw = e_pad // NW
    nb = epw // BE
    vpr = h // L

    @functools.partial(
        pl.kernel,
        out_type=jax.ShapeDtypeStruct((e_pad, h), f32),
        mesh=_mesh(),
        scratch_types=[
            pltpu.VMEM((epw,), jnp.int32),
            pltpu.VMEM((epw,), jnp.int32),
            pltpu.VMEM((BE, h), f32),
            pltpu.VMEM((BE, h), f32),
            pltpu.VMEM((BE, h), f32),
            pltpu.SemaphoreType.DMA,
        ],
    )
    def gather_kernel(ps_h, pd_h, src_h, dst_h, c_h, g_h, si, di, rs, rd, cc, sem):
        base = _wid() * epw
        pltpu.sync_copy(src_h.at[pl.ds(base, epw)], si)
        pltpu.sync_copy(dst_h.at[pl.ds(base, epw)], di)
        for b in range(nb):
            eb = base + b * BE
            pltpu.sync_copy(c_h.at[pl.ds(eb, BE)], cc)
            cps = []
            for k in range(BE // 128):
                o = b * BE + k * 128
                cps.append(
                    pltpu.async_copy(
                        ps_h.at[si.at[pl.ds(o, 128)]], rs.at[pl.ds(k * 128, 128)], sem
                    )
                )
                cps.append(
                    pltpu.async_copy(
                        pd_h.at[di.at[pl.ds(o, 128)]], rd.at[pl.ds(k * 128, 128)], sem
                    )
                )
            for cp in cps:
                cp.wait()

            @plsc.parallel_loop(0, BE, unroll=4)
            def _(r):
                for j in range(vpr):
                    c0 = j * L
                    rs[r, pl.ds(c0, L)] = (
                        rs[r, pl.ds(c0, L)] + rd[r, pl.ds(c0, L)] + cc[r, pl.ds(c0, L)]
                    )

            pltpu.sync_copy(rs, g_h.at[pl.ds(eb, BE)])

    return gather_kernel


def _make_scatter_kernel(n_pad, e_pad, h):
    epw = e_pad // NW
    nb = epw // BE
    nchunk = BE // 128
    rows_pt = n_pad // NS

    @functools.partial(
        pl.kernel,
        out_type=[
            jax.ShapeDtypeStruct((n_pad, h), f32),
            jax.ShapeDtypeStruct((n_pad, h), f32),
        ],
        mesh=_mesh(),
        scratch_types=[
            pltpu.VMEM((epw // 128, 128), jnp.int32),
            pltpu.VMEM((BE, h), f32),
            pltpu.VMEM_SHARED((n_pad, h), f32),
        ],
    )
    def scatter_kernel(m_h, dsti_h, o0, o1, di, rows, agg):
        cid = lax.axis_index("c")
        sid = lax.axis_index("s")
        wid = sid * NC + cid
        base = wid * epw
        pltpu.sync_copy(dsti_h.at[pl.ds(wid * (epw // 128), epw // 128)], di)

        vpr = h // L

        @plsc.parallel_loop(0, rows_pt * vpr, unroll=4)
        def _(i):
            r = i // vpr
            c0 = (i % vpr) * L
            rows[r, pl.ds(c0, L)] = jnp.zeros((L,), f32)

        pltpu.sync_copy(
            rows.at[pl.ds(0, rows_pt)], agg.at[pl.ds(sid * rows_pt, rows_pt)]
        )
        plsc.subcore_barrier()

        for b in range(nb):
            eb = base + b * BE
            pltpu.sync_copy(m_h.at[pl.ds(eb, BE)], rows)
            for k in range(nchunk):
                pltpu.sync_copy(
                    rows.at[pl.ds(k * 128, 128)],
                    agg.at[di.at[b * nchunk + k]],
                    add=True,
                )
        plsc.subcore_barrier()

        @pl.when(cid == 0)
        def _():
            pltpu.sync_copy(
                agg.at[pl.ds(sid * rows_pt, rows_pt)],
                o0.at[pl.ds(sid * rows_pt, rows_pt)],
            )

        @pl.when(cid == 1)
        def _():
            pltpu.sync_copy(
                agg.at[pl.ds(sid * rows_pt, rows_pt)],
                o1.at[pl.ds(sid * rows_pt, rows_pt)],
            )

    return scatter_kernel


def _dot(a, b):
    return jnp.dot(a, b, preferred_element_type=f32)


def _tc_inject(xp, W, b, blk=2048):
    n_pad, d = xp.shape

    def body(x_r, w_r, b_r, o_r):
        o_r[...] = _dot(x_r[...], w_r[...]) + b_r[...]

    return pl.pallas_call(
        body,
        grid=(n_pad // blk,),
        in_specs=[
            pl.BlockSpec((blk, d), lambda i: (i, 0)),
            pl.BlockSpec((d, d), lambda i: (0, 0)),
            pl.BlockSpec((1, d), lambda i: (0, 0)),
        ],
        out_specs=pl.BlockSpec((blk, d), lambda i: (i, 0)),
        out_shape=jax.ShapeDtypeStruct((n_pad, d), f32),
    )(xp, W, b)


def _tc_edge_const(rad4, eap, R1, P1, b1, R2, P2, b2, blk=512):
    ne, four = rad4.shape
    dea = eap.shape[1]
    hp = R1.shape[1]

    def body(r_r, e_r, R1r, P1r, b1r, R2r, P2r, b2r, c1r, c2r):
        r = r_r[...]
        ea = e_r[...]
        c1r[...] = _dot(r, R1r[...]) + _dot(ea, P1r[...]) + b1r[...]
        c2r[...] = _dot(r, R2r[...]) + _dot(ea, P2r[...]) + b2r[...]

    return pl.pallas_call(
        body,
        grid=(ne // blk,),
        in_specs=[
            pl.BlockSpec((blk, four), lambda i: (i, 0)),
            pl.BlockSpec((blk, dea), lambda i: (i, 0)),
            pl.BlockSpec((four, hp), lambda i: (0, 0)),
            pl.BlockSpec((dea, hp), lambda i: (0, 0)),
            pl.BlockSpec((1, hp), lambda i: (0, 0)),
            pl.BlockSpec((four, hp), lambda i: (0, 0)),
            pl.BlockSpec((dea, hp), lambda i: (0, 0)),
            pl.BlockSpec((1, hp), lambda i: (0, 0)),
        ],
        out_specs=[
            pl.BlockSpec((blk, hp), lambda i: (i, 0)),
            pl.BlockSpec((blk, hp), lambda i: (i, 0)),
        ],
        out_shape=[
            jax.ShapeDtypeStruct((ne, hp), f32),
            jax.ShapeDtypeStruct((ne, hp), f32),
        ],
    )(rad4, eap, R1, P1, b1, R2, P2, b2)


def _tc_edge_mlp(gp, W2k, b2t, blk=512):
    ne, hp = gp.shape

    def body(g_r, w_r, b_r, o_r):
        m = _gelu(g_r[...])
        o_r[...] = _gelu(_dot(m, w_r[...]) + b_r[...])

    return pl.pallas_call(
        body,
        grid=(ne // blk,),
        in_specs=[
            pl.BlockSpec((blk, hp), lambda i: (i, 0)),
            pl.BlockSpec((hp, hp), lambda i: (0, 0)),
            pl.BlockSpec((1, hp), lambda i: (0, 0)),
        ],
        out_specs=pl.BlockSpec((blk, hp), lambda i: (i, 0)),
        out_shape=jax.ShapeDtypeStruct((ne, hp), f32),
    )(gp, W2k, b2t)


def _tc_node(hst, a0, a1, res, Wh, Wagg, nb_, W1s, W1d, blk=2048):
    n_pad, d = hst.shape
    h = a0.shape[1]

    def body(h_r, a0_r, a1_r, res_r, Wh_r, Wa_r, nb_r, W1s_r, W1d_r, hn_r, ps_r, pd_r):
        z = _dot(h_r[...], Wh_r[...]) + _dot(a0_r[...] + a1_r[...], Wa_r[...]) + nb_r[...]
        hn = _gelu(z + res_r[...])
        hn_r[...] = hn
        ps_r[...] = _dot(hn, W1s_r[...])
        pd_r[...] = _dot(hn, W1d_r[...])

    return pl.pallas_call(
        body,
        grid=(n_pad // blk,),
        in_specs=[
            pl.BlockSpec((blk, d), lambda i: (i, 0)),
            pl.BlockSpec((blk, h), lambda i: (i, 0)),
            pl.BlockSpec((blk, h), lambda i: (i, 0)),
            pl.BlockSpec((blk, d), lambda i: (i, 0)),
            pl.BlockSpec((d, d), lambda i: (0, 0)),
            pl.BlockSpec((h, d), lambda i: (0, 0)),
            pl.BlockSpec((1, d), lambda i: (0, 0)),
            pl.BlockSpec((d, h), lambda i: (0, 0)),
            pl.BlockSpec((d, h), lambda i: (0, 0)),
        ],
        out_specs=[
            pl.BlockSpec((blk, d), lambda i: (i, 0)),
            pl.BlockSpec((blk, h), lambda i: (i, 0)),
            pl.BlockSpec((blk, h), lambda i: (i, 0)),
        ],
        out_shape=[
            jax.ShapeDtypeStruct((n_pad, d), f32),
            jax.ShapeDtypeStruct((n_pad, h), f32),
            jax.ShapeDtypeStruct((n_pad, h), f32),
        ],
    )(hst, a0, a1, res, Wh, Wagg, nb_, W1s, W1d)


def kernel(x, pos, edge_index, edge_attr, params):
    n, d = x.shape
    e = edge_index.shape[1]
    de = edge_attr.shape[1]
    h = params["egnn1"]["eW2"].shape[0]

    chunk_e = NW * BE
    e_pad = ((e + chunk_e - 1) // chunk_e) * chunk_e
    blk_n = 2048
    n_pad = ((n + blk_n - 1) // blk_n) * blk_n

    src = edge_index[0]
    dst = edge_index[1]
    pe = e_pad - e
    srcp = jnp.concatenate([src, jnp.zeros((pe,), jnp.int32)])
    dstg = jnp.concatenate([dst, jnp.zeros((pe,), jnp.int32)])
    dsts = jnp.concatenate([dst, jnp.full((pe,), n, jnp.int32)])
    eap = jnp.concatenate([edge_attr, jnp.zeros((pe, de), f32)]).reshape(e_pad // 4, 4 * de)
    xp = jnp.concatenate([x, jnp.zeros((n_pad - n, d), f32)])

    eye4 = jnp.eye(4, dtype=f32)

    def prep(p):
        eW1 = p["eW1"]
        return dict(
            W1s=eW1[:d],
            W1d=eW1[d : 2 * d],
            Rk=jnp.kron(eye4, eW1[2 * d : 2 * d + 1]),
            Pk=jnp.kron(eye4, eW1[2 * d + 1 :]),
            b1t=jnp.tile(p["eb1"], 4)[None],
            W2k=jnp.kron(eye4, p["eW2"]),
            b2t=jnp.tile(p["eb2"], 4)[None],
            Wh=p["nW"][:d],
            Wagg=p["nW"][d:],
            nb=p["nb"][None],
        )

    w1 = prep(params["egnn1"])
    w2 = prep(params["egnn2"])

    rad_k = _make_rad_kernel(n, e_pad)
    gather_k = _make_gather_kernel(n_pad, e_pad, h)
    scatter_k = _make_scatter_kernel(n_pad, e_pad, h)

    x_inj = _tc_inject(xp, params["inj_W"], params["inj_b"][None])
    rad = rad_k(pos[:, 0], pos[:, 1], pos[:, 2], srcp, dstg)
    c1p, c2p = _tc_edge_const(
        rad.reshape(e_pad // 4, 4), eap,
        w1["Rk"], w1["Pk"], w1["b1t"], w2["Rk"], w2["Pk"], w2["b1t"],
    )
    c1 = c1p.reshape(e_pad, h)
    c2 = c2p.reshape(e_pad, h)
    dsti = dsts.reshape(e_pad // 128, 128)

    z = jnp.zeros((n_pad, d), f32)
    ps = jnp.zeros((n_pad, h), f32)
    pd = jnp.zeros((n_pad, h), f32)

    for _ in range(ITERS):
        z0 = z
        for li, (w, wn, c) in enumerate(((w1, w2, c1), (w2, w1, c2))):
            g = gather_k(ps, pd, srcp, dstg, c)
            m = _tc_edge_mlp(g.reshape(e_pad // 4, 4 * h), w["W2k"], w["b2t"])
            a0, a1 = scatter_k(m.reshape(e_pad, h), dsti)
            res = x_inj if li == 0 else z0
            z, ps, pd = _tc_node(
                z, a0, a1, res, w["Wh"], w["Wagg"], w["nb"], wn["W1s"], wn["W1d"]
            )
    return z[:n]


# speedup vs baseline: 4.4670x; 4.4670x over previous
import functools

import jax
import jax.numpy as jnp
from jax import lax
from jax.experimental import pallas as pl
from jax.experimental.pallas import tpu as pltpu
from jax.experimental.pallas import tpu_sc as plsc

NC = 2
NS = 16
L = 16
NW = NC * NS
BE = 1024
ITERS = 6

f32 = jnp.float32


def _gelu(v):
    return jax.nn.gelu(v, approximate=True)


def _mesh():
    return plsc.VectorSubcoreMesh(
        core_axis_name="c", subcore_axis_name="s", num_cores=NC, num_subcores=NS
    )


_SC_PARAMS = pltpu.CompilerParams(use_tc_tiling_on_sc=False)


def _wid():
    return lax.axis_index("s") * NC + lax.axis_index("c")


def _make_rad_kernel(n, e_pad):
    epw = e_pad // NW
    nb = epw // BE

    @functools.partial(
        pl.kernel,
        out_type=jax.ShapeDtypeStruct((e_pad, L), f32),
        mesh=_mesh(),
        compiler_params=_SC_PARAMS,
        scratch_types=[
            pltpu.VMEM((epw,), jnp.int32),
            pltpu.VMEM((epw,), jnp.int32),
            pltpu.VMEM((BE, L), f32),
            pltpu.VMEM((BE, L), f32),
            pltpu.SemaphoreType.DMA,
        ],
    )
    def rad_kernel(p16_h, src_h, dst_h, r_h, si, di, rs, rd, sem):
        base = _wid() * epw
        pltpu.sync_copy(src_h.at[pl.ds(base, epw)], si)
        pltpu.sync_copy(dst_h.at[pl.ds(base, epw)], di)
        for b in range(nb):
            eb = base + b * BE
            cps = []
            for k in range(BE // 128):
                o = b * BE + k * 128
                cps.append(
                    pltpu.async_copy(
                        p16_h.at[si.at[pl.ds(o, 128)]], rs.at[pl.ds(k * 128, 128)], sem
                    )
                )
                cps.append(
                    pltpu.async_copy(
                        p16_h.at[di.at[pl.ds(o, 128)]], rd.at[pl.ds(k * 128, 128)], sem
                    )
                )
            for cp in cps:
                cp.wait()

            @plsc.parallel_loop(0, BE, unroll=4)
            def _(r):
                dv = rs[r, pl.ds(0, L)] - rd[r, pl.ds(0, L)]
                rs[r, pl.ds(0, L)] = dv * dv

            pltpu.sync_copy(rs, r_h.at[pl.ds(eb, BE)])

    return rad_kernel


def _make_gather_kernel(n_pad, e_pad, h):
    epw = e_pad // NW
    nb = epw // BE
    vpr = h // L

    @functools.partial(
        pl.kernel,
        out_type=jax.ShapeDtypeStruct((e_pad, h), f32),
        mesh=_mesh(),
        compiler_params=_SC_PARAMS,
        scratch_types=[
            pltpu.VMEM((epw,), jnp.int32),
            pltpu.VMEM((epw,), jnp.int32),
            pltpu.VMEM((BE, h), f32),
            pltpu.VMEM((BE, h), f32),
            pltpu.VMEM((BE, h), f32),
            pltpu.SemaphoreType.DMA,
        ],
    )
    def gather_kernel(ps_h, pd_h, src_h, dst_h, c_h, g_h, si, di, rs, rd, cc, sem):
        base = _wid() * epw
        pltpu.sync_copy(src_h.at[pl.ds(base, epw)], si)
        pltpu.sync_copy(dst_h.at[pl.ds(base, epw)], di)
        for b in range(nb):
            eb = base + b * BE
            pltpu.sync_copy(c_h.at[pl.ds(eb, BE)], cc)
            cps = []
            for k in range(BE // 128):
                o = b * BE + k * 128
                cps.append(
                    pltpu.async_copy(
                        ps_h.at[si.at[pl.ds(o, 128)]], rs.at[pl.ds(k * 128, 128)], sem
                    )
                )
                cps.append(
                    pltpu.async_copy(
                        pd_h.at[di.at[pl.ds(o, 128)]], rd.at[pl.ds(k * 128, 128)], sem
                    )
                )
            for cp in cps:
                cp.wait()

            @plsc.parallel_loop(0, BE, unroll=4)
            def _(r):
                for j in range(vpr):
                    c0 = j * L
                    rs[r, pl.ds(c0, L)] = (
                        rs[r, pl.ds(c0, L)] + rd[r, pl.ds(c0, L)] + cc[r, pl.ds(c0, L)]
                    )

            pltpu.sync_copy(rs, g_h.at[pl.ds(eb, BE)])

    return gather_kernel


def _make_scatter_kernel(n_pad, e_pad, h):
    epw = e_pad // NW
    nb = epw // BE
    nchunk = BE // 128
    rows_pt = n_pad // NS

    @functools.partial(
        pl.kernel,
        out_type=[
            jax.ShapeDtypeStruct((n_pad, h), f32),
            jax.ShapeDtypeStruct((n_pad, h), f32),
        ],
        mesh=_mesh(),
        compiler_params=_SC_PARAMS,
        scratch_types=[
            pltpu.VMEM((epw // 128, 128), jnp.int32),
            pltpu.VMEM((BE, h), f32),
            pltpu.VMEM_SHARED((n_pad, h), f32),
        ],
    )
    def scatter_kernel(m_h, dsti_h, o0, o1, di, rows, agg):
        cid = lax.axis_index("c")
        sid = lax.axis_index("s")
        wid = sid * NC + cid
        base = wid * epw
        pltpu.sync_copy(dsti_h.at[pl.ds(wid * (epw // 128), epw // 128)], di)

        vpr = h // L

        @plsc.parallel_loop(0, rows_pt * vpr, unroll=4)
        def _(i):
            r = i // vpr
            c0 = (i % vpr) * L
            rows[r, pl.ds(c0, L)] = jnp.zeros((L,), f32)

        pltpu.sync_copy(
            rows.at[pl.ds(0, rows_pt)], agg.at[pl.ds(sid * rows_pt, rows_pt)]
        )
        plsc.subcore_barrier()

        for b in range(nb):
            eb = base + b * BE
            pltpu.sync_copy(m_h.at[pl.ds(eb, BE)], rows)
            for k in range(nchunk):
                pltpu.sync_copy(
                    rows.at[pl.ds(k * 128, 128)],
                    agg.at[di.at[b * nchunk + k]],
                    add=True,
                )
        plsc.subcore_barrier()

        @pl.when(cid == 0)
        def _():
            pltpu.sync_copy(
                agg.at[pl.ds(sid * rows_pt, rows_pt)],
                o0.at[pl.ds(sid * rows_pt, rows_pt)],
            )

        @pl.when(cid == 1)
        def _():
            pltpu.sync_copy(
                agg.at[pl.ds(sid * rows_pt, rows_pt)],
                o1.at[pl.ds(sid * rows_pt, rows_pt)],
            )

    return scatter_kernel


def _dot(a, b):
    return jnp.dot(
        a.astype(jnp.bfloat16), b.astype(jnp.bfloat16), preferred_element_type=f32
    )


def _tc_inject(xp, W, b, blk=2048):
    n_pad, d = xp.shape

    def body(x_r, w_r, b_r, o_r):
        o_r[...] = _dot(x_r[...], w_r[...]) + b_r[...]

    return pl.pallas_call(
        body,
        grid=(n_pad // blk,),
        in_specs=[
            pl.BlockSpec((blk, d), lambda i: (i, 0)),
            pl.BlockSpec((d, d), lambda i: (0, 0)),
            pl.BlockSpec((1, d), lambda i: (0, 0)),
        ],
        out_specs=pl.BlockSpec((blk, d), lambda i: (i, 0)),
        out_shape=jax.ShapeDtypeStruct((n_pad, d), f32),
    )(xp, W, b)


def _tc_edge_const(r16, eap, wr1, P1, b1, wr2, P2, b2, blk=2048):
    ne, lanes = r16.shape
    dea = eap.shape[1]
    hp = P1.shape[1]

    def body(r_r, e_r, w1r, P1r, b1r, w2r, P2r, b2r, c1r, c2r):
        rad = jnp.sum(r_r[...], axis=1, keepdims=True)
        radb = rad.astype(jnp.bfloat16).astype(f32)
        ea = e_r[...]
        wr1b = w1r[...].astype(jnp.bfloat16).astype(f32)
        wr2b = w2r[...].astype(jnp.bfloat16).astype(f32)
        c1r[...] = radb * wr1b + _dot(ea, P1r[...]) + b1r[...]
        c2r[...] = radb * wr2b + _dot(ea, P2r[...]) + b2r[...]

    return pl.pallas_call(
        body,
        grid=(ne // blk,),
        in_specs=[
            pl.BlockSpec((blk, lanes), lambda i: (i, 0)),
            pl.BlockSpec((blk, dea), lambda i: (i, 0)),
            pl.BlockSpec((1, hp), lambda i: (0, 0)),
            pl.BlockSpec((dea, hp), lambda i: (0, 0)),
            pl.BlockSpec((1, hp), lambda i: (0, 0)),
            pl.BlockSpec((1, hp), lambda i: (0, 0)),
            pl.BlockSpec((dea, hp), lambda i: (0, 0)),
            pl.BlockSpec((1, hp), lambda i: (0, 0)),
        ],
        out_specs=[
            pl.BlockSpec((blk, hp), lambda i: (i, 0)),
            pl.BlockSpec((blk, hp), lambda i: (i, 0)),
        ],
        out_shape=[
            jax.ShapeDtypeStruct((ne, hp), f32),
            jax.ShapeDtypeStruct((ne, hp), f32),
        ],
    )(r16, eap, wr1, P1, b1, wr2, P2, b2)


def _tc_edge_mlp(gp, W2k, b2t, blk=512):
    ne, hp = gp.shape

    def body(g_r, w_r, b_r, o_r):
        m = _gelu(g_r[...])
        o_r[...] = _gelu(_dot(m, w_r[...]) + b_r[...])

    return pl.pallas_call(
        body,
        grid=(ne // blk,),
        in_specs=[
            pl.BlockSpec((blk, hp), lambda i: (i, 0)),
            pl.BlockSpec((hp, hp), lambda i: (0, 0)),
            pl.BlockSpec((1, hp), lambda i: (0, 0)),
        ],
        out_specs=pl.BlockSpec((blk, hp), lambda i: (i, 0)),
        out_shape=jax.ShapeDtypeStruct((ne, hp), f32),
    )(gp, W2k, b2t)


def _tc_node(hst, a0, a1, res, Wh, Wagg, nb_, W1s, W1d, blk=2048):
    n_pad, d = hst.shape
    h = a0.shape[1]

    def body(h_r, a0_r, a1_r, res_r, Wh_r, Wa_r, nb_r, W1s_r, W1d_r, hn_r, ps_r, pd_r):
        z = _dot(h_r[...], Wh_r[...]) + _dot(a0_r[...] + a1_r[...], Wa_r[...]) + nb_r[...]
        hn = _gelu(z + res_r[...])
        hn_r[...] = hn
        ps_r[...] = _dot(hn, W1s_r[...])
        pd_r[...] = _dot(hn, W1d_r[...])

    return pl.pallas_call(
        body,
        grid=(n_pad // blk,),
        in_specs=[
            pl.BlockSpec((blk, d), lambda i: (i, 0)),
            pl.BlockSpec((blk, h), lambda i: (i, 0)),
            pl.BlockSpec((blk, h), lambda i: (i, 0)),
            pl.BlockSpec((blk, d), lambda i: (i, 0)),
            pl.BlockSpec((d, d), lambda i: (0, 0)),
            pl.BlockSpec((h, d), lambda i: (0, 0)),
            pl.BlockSpec((1, d), lambda i: (0, 0)),
            pl.BlockSpec((d, h), lambda i: (0, 0)),
            pl.BlockSpec((d, h), lambda i: (0, 0)),
        ],
        out_specs=[
            pl.BlockSpec((blk, d), lambda i: (i, 0)),
            pl.BlockSpec((blk, h), lambda i: (i, 0)),
            pl.BlockSpec((blk, h), lambda i: (i, 0)),
        ],
        out_shape=[
            jax.ShapeDtypeStruct((n_pad, d), f32),
            jax.ShapeDtypeStruct((n_pad, h), f32),
            jax.ShapeDtypeStruct((n_pad, h), f32),
        ],
    )(hst, a0, a1, res, Wh, Wagg, nb_, W1s, W1d)


def kernel(x, pos, edge_index, edge_attr, params):
    n, d = x.shape
    e = edge_index.shape[1]
    de = edge_attr.shape[1]
    h = params["egnn1"]["eW2"].shape[0]

    chunk_e = NW * BE
    e_pad = ((e + chunk_e - 1) // chunk_e) * chunk_e
    blk_n = 2048
    n_pad = ((n + blk_n - 1) // blk_n) * blk_n

    src = edge_index[0]
    dst = edge_index[1]
    pe = e_pad - e
    srcp = jnp.concatenate([src, jnp.zeros((pe,), jnp.int32)])
    dstg = jnp.concatenate([dst, jnp.zeros((pe,), jnp.int32)])
    dsts = jnp.concatenate([dst, jnp.full((pe,), n, jnp.int32)])
    eap = jnp.concatenate([edge_attr, jnp.zeros((pe, de), f32)])
    xp = jnp.concatenate([x, jnp.zeros((n_pad - n, d), f32)])
    p16 = jnp.concatenate([pos, jnp.zeros((n, L - pos.shape[1]), f32)], axis=1)

    eye4 = jnp.eye(4, dtype=f32)

    def prep(p):
        eW1 = p["eW1"]
        return dict(
            W1s=eW1[:d],
            W1d=eW1[d : 2 * d],
            wr=eW1[2 * d : 2 * d + 1],
            W1e=eW1[2 * d + 1 :],
            b1=p["eb1"][None],
            W2k=jnp.kron(eye4, p["eW2"]),
            b2t=jnp.tile(p["eb2"], 4)[None],
            Wh=p["nW"][:d],
            Wagg=p["nW"][d:],
            nb=p["nb"][None],
        )

    w1 = prep(params["egnn1"])
    w2 = prep(params["egnn2"])

    rad_k = _make_rad_kernel(n, e_pad)
    gather_k = _make_gather_kernel(n_pad, e_pad, h)
    scatter_k = _make_scatter_kernel(n_pad, e_pad, h)

    x_inj = _tc_inject(xp, params["inj_W"], params["inj_b"][None])
    r16 = rad_k(p16, srcp, dstg)
    c1, c2 = _tc_edge_const(
        r16, eap,
        w1["wr"], w1["W1e"], w1["b1"], w2["wr"], w2["W1e"], w2["b1"],
    )
    dsti = dsts.reshape(e_pad // 128, 128)

    z = jnp.zeros((n_pad, d), f32)
    ps = jnp.zeros((n_pad, h), f32)
    pd = jnp.zeros((n_pad, h), f32)

    for _ in range(ITERS):
        z0 = z
        for li, (w, wn, c) in enumerate(((w1, w2, c1), (w2, w1, c2))):
            g = gather_k(ps, pd, srcp, dstg, c)
            m = _tc_edge_mlp(g.reshape(e_pad // 4, 4 * h), w["W2k"], w["b2t"])
            a0, a1 = scatter_k(m.reshape(e_pad, h), dsti)
            res = x_inj if li == 0 else z0
            z, ps, pd = _tc_node(
                z, a0, a1, res, w["Wh"], w["Wagg"], w["nb"], wn["W1s"], wn["W1d"]
            )
    return z[:n]
